# baseline (speedup 1.0000x reference)
import dataclasses
import functools

import jax
import jax.numpy as jnp
from jax import lax
from jax.experimental import pallas as pl
from jax.experimental.pallas import tpu as pltpu
from jax.experimental.pallas import tpu_sc as plsc

_NC = 2
_NS = 16
_NW = _NC * _NS
_L = 16


def _sc_hist(idx_flat, num_experts):
    n = idx_flat.shape[0]
    per_w = n // _NW
    vecs = per_w // _L
    mesh = plsc.VectorSubcoreMesh(core_axis_name="c", subcore_axis_name="s")
    cp = pltpu.CompilerParams()
    if "needs_layout_passes" in pltpu.CompilerParams.__dataclass_fields__:
        cp = dataclasses.replace(cp, needs_layout_passes=False)

    @functools.partial(
        pl.kernel,
        mesh=mesh,
        compiler_params=cp,
        out_type=jax.ShapeDtypeStruct((_NW * _L * num_experts,), jnp.float32),
        scratch_types=[
            pltpu.VMEM((per_w,), jnp.int32),
            pltpu.VMEM((_L * num_experts,), jnp.float32),
            pltpu.SemaphoreType.DMA,
        ],
    )
    def hist_kernel(idx_hbm, out_hbm, idx_v, hist_v, sem):
        c = lax.axis_index("c")
        s = lax.axis_index("s")
        wid = s * _NC + c
        base = wid * per_w
        cp = pltpu.async_copy(idx_hbm.at[pl.ds(base, per_w)], idx_v, sem)

        zeros = jnp.zeros((_L,), jnp.float32)
        for j in range(num_experts):
            hist_v[pl.ds(j * _L, _L)] = zeros

        lane_off = lax.broadcasted_iota(jnp.int32, (_L,), 0) * num_experts
        ones = jnp.ones((_L,), jnp.float32)
        cp.wait()

        @pl.loop(0, vecs)
        def _(i):
            e = idx_v[pl.ds(i * _L, _L)]
            plsc.addupdate_scatter(hist_v, [lane_off + e], ones)

        pltpu.sync_copy(hist_v, out_hbm.at[pl.ds(wid * _L * num_experts,
                                                 _L * num_experts)])

    return hist_kernel(idx_flat)


def _tc_body(x_ref, c_ref, out_ref, acc_p, acc_c, *, inv_tokens, inv_assign,
             num_experts):
    i = pl.program_id(0)

    @pl.when(i == 0)
    def _():
        acc_p[...] = jnp.zeros_like(acc_p)
        acc_c[...] = jnp.zeros_like(acc_c)

    x = x_ref[...]
    m = jnp.max(x, axis=1, keepdims=True)
    e = jnp.exp(x - m)
    p = e / jnp.sum(e, axis=1, keepdims=True)
    acc_p[...] += jnp.sum(p, axis=0, keepdims=True)
    acc_c[...] += jnp.sum(c_ref[...], axis=0, keepdims=True)

    @pl.when(i == pl.num_programs(0) - 1)
    def _():
        avg_prob = acc_p[...] * inv_tokens
        frac = acc_c[...] * inv_assign
        out_ref[0] = 0.01 * num_experts * jnp.sum(frac * avg_prob)


def _tc_loss(x2d, counts, total_assign):
    tokens, num_experts = x2d.shape
    steps = 16
    rows = tokens // steps
    crows = counts.shape[0] // steps
    body = functools.partial(
        _tc_body,
        inv_tokens=1.0 / tokens,
        inv_assign=1.0 / total_assign,
        num_experts=float(num_experts),
    )
    return pl.pallas_call(
        body,
        grid=(steps,),
        in_specs=[
            pl.BlockSpec((rows, num_experts), lambda i: (i, 0)),
            pl.BlockSpec((crows, num_experts), lambda i: (i, 0)),
        ],
        out_specs=pl.BlockSpec(memory_space=pltpu.SMEM),
        out_shape=jax.ShapeDtypeStruct((1,), jnp.float32),
        scratch_shapes=[
            pltpu.VMEM((1, num_experts), jnp.float32),
            pltpu.VMEM((1, num_experts), jnp.float32),
        ],
    )(x2d, counts)


def kernel(router_logits, expert_indices):
    batch, seq, num_experts = router_logits.shape
    k = expert_indices.shape[-1]
    tokens = batch * seq
    total_assign = tokens * k

    x2d = router_logits.reshape(tokens, num_experts)
    idx_flat = expert_indices.reshape(-1).astype(jnp.int32)

    counts_flat = _sc_hist(idx_flat, num_experts)
    counts = counts_flat.reshape(_NW * _L, num_experts)

    loss = _tc_loss(x2d, counts, total_assign)
    return loss[0]


# speedup vs baseline: 1.1649x; 1.1649x over previous
import dataclasses
import functools

import jax
import jax.numpy as jnp
from jax import lax
from jax.experimental import pallas as pl
from jax.experimental.pallas import tpu as pltpu
from jax.experimental.pallas import tpu_sc as plsc

_NC = 2
_NS = 16
_NW = _NC * _NS
_L = 16


def _softmax_flatten_body(x_ref, i_ref, psum_ref, idxt_ref, acc_ref):
    i = pl.program_id(0)

    @pl.when(i == 0)
    def _():
        acc_ref[...] = jnp.zeros_like(acc_ref)

    x = x_ref[...]
    m = jnp.max(x, axis=1, keepdims=True)
    e = jnp.exp(x - m)
    r = 1.0 / jnp.sum(e, axis=1, keepdims=True)
    acc_ref[...] += jnp.sum(e * r, axis=0, keepdims=True)

    idxt_ref[...] = i_ref[...].T

    @pl.when(i == pl.num_programs(0) - 1)
    def _():
        psum_ref[...] = acc_ref[...]


def _softmax_flatten(x2d, idx2d):
    tokens, num_experts = x2d.shape
    k = idx2d.shape[-1]
    steps = 16
    rows = tokens // steps
    return pl.pallas_call(
        _softmax_flatten_body,
        grid=(steps,),
        in_specs=[
            pl.BlockSpec((rows, num_experts), lambda i: (i, 0)),
            pl.BlockSpec((rows, k), lambda i: (i, 0)),
        ],
        out_specs=[
            pl.BlockSpec((1, num_experts), lambda i: (0, 0)),
            pl.BlockSpec((k, rows), lambda i: (0, i)),
        ],
        out_shape=[
            jax.ShapeDtypeStruct((1, num_experts), jnp.float32),
            jax.ShapeDtypeStruct((k, tokens), jnp.int32),
        ],
        scratch_shapes=[pltpu.VMEM((1, num_experts), jnp.float32)],
    )(x2d, idx2d)


def _sc_hist(idxt, num_experts):
    n = idxt.shape[0] * idxt.shape[1]
    rows = idxt.shape[0]
    chunks = _NW // rows
    width = idxt.shape[1] // chunks
    vecs = width // _L
    mesh = plsc.VectorSubcoreMesh(core_axis_name="c", subcore_axis_name="s")
    cp = pltpu.CompilerParams()
    if "needs_layout_passes" in pltpu.CompilerParams.__dataclass_fields__:
        cp = dataclasses.replace(cp, needs_layout_passes=False)

    @functools.partial(
        pl.kernel,
        mesh=mesh,
        compiler_params=cp,
        out_type=jax.ShapeDtypeStruct((_NW * _L * num_experts,), jnp.float32),
        scratch_types=[
            pltpu.VMEM((width,), jnp.int32),
            pltpu.VMEM((_L * num_experts,), jnp.float32),
            pltpu.SemaphoreType.DMA,
        ],
    )
    def hist_kernel(idx_hbm, out_hbm, idx_v, hist_v, sem):
        c = lax.axis_index("c")
        s = lax.axis_index("s")
        wid = s * _NC + c
        row = wid // chunks
        col0 = (wid % chunks) * width
        cpy = pltpu.async_copy(idx_hbm.at[row, pl.ds(col0, width)], idx_v, sem)

        zeros = jnp.zeros((_L,), jnp.float32)
        for j in range(num_experts):
            hist_v[pl.ds(j * _L, _L)] = zeros

        lane_off = lax.broadcasted_iota(jnp.int32, (_L,), 0) * num_experts
        ones = jnp.ones((_L,), jnp.float32)
        cpy.wait()

        @pl.loop(0, vecs, step=8)
        def _(i):
            for j in range(8):
                e = idx_v[pl.ds((i + j) * _L, _L)]
                plsc.addupdate_scatter(hist_v, [lane_off + e], ones)

        pltpu.sync_copy(hist_v, out_hbm.at[pl.ds(wid * _L * num_experts,
                                                 _L * num_experts)])

    return hist_kernel(idxt)


def _combine_body(psum_ref, cnt_ref, out_ref, *, inv_tokens, inv_assign,
                  num_experts):
    counts = jnp.sum(cnt_ref[...], axis=0, keepdims=True)
    avg_prob = psum_ref[...] * inv_tokens
    frac = counts * inv_assign
    out_ref[0] = 0.01 * num_experts * jnp.sum(frac * avg_prob)


def _combine(psum, counts, tokens, total_assign):
    num_experts = psum.shape[-1]
    body = functools.partial(
        _combine_body,
        inv_tokens=1.0 / tokens,
        inv_assign=1.0 / total_assign,
        num_experts=float(num_experts),
    )
    return pl.pallas_call(
        body,
        out_specs=pl.BlockSpec(memory_space=pltpu.SMEM),
        out_shape=jax.ShapeDtypeStruct((1,), jnp.float32),
    )(psum, counts)


def kernel(router_logits, expert_indices):
    batch, seq, num_experts = router_logits.shape
    k = expert_indices.shape[-1]
    tokens = batch * seq
    total_assign = tokens * k

    x2d = router_logits.reshape(tokens, num_experts)
    idx2d = expert_indices.reshape(tokens, k).astype(jnp.int32)

    psum, idxt = _softmax_flatten(x2d, idx2d)
    counts_flat = _sc_hist(idxt, num_experts)
    counts = counts_flat.reshape(_NW * _L, num_experts)

    loss = _combine(psum, counts, tokens, total_assign)
    return loss[0]


# speedup vs baseline: 2.7115x; 2.3277x over previous
import dataclasses
import functools

import jax
import jax.numpy as jnp
from jax import lax
from jax.experimental import pallas as pl
from jax.experimental.pallas import tpu as pltpu
from jax.experimental.pallas import tpu_sc as plsc

_NC = 2
_NS = 16
_NW = _NC * _NS
_L = 16


def _softmax_body(x_ref, psum_ref, acc_ref):
    i = pl.program_id(0)
    j = pl.program_id(1)

    @pl.when((i == 0) & (j == 0))
    def _():
        acc_ref[...] = jnp.zeros_like(acc_ref)

    x = x_ref[...]
    m = jnp.max(x, axis=0, keepdims=True)
    e = jnp.exp(x - m)
    r = 1.0 / jnp.sum(e, axis=0, keepdims=True)
    acc_ref[...] += e * r

    @pl.when((i == pl.num_programs(0) - 1) & (j == pl.num_programs(1) - 1))
    def _():
        psum_ref[...] = jnp.sum(acc_ref[...], axis=1, keepdims=True)


def _softmax_psum(xT, num_experts):
    rows, seq = xT.shape
    batch = rows // num_experts
    blk = 4096
    return pl.pallas_call(
        _softmax_body,
        grid=(batch, seq // blk),
        in_specs=[pl.BlockSpec((num_experts, blk), lambda i, j: (i, j))],
        out_specs=pl.BlockSpec((num_experts, 1), lambda i, j: (0, 0)),
        out_shape=jax.ShapeDtypeStruct((num_experts, 1), jnp.float32),
        scratch_shapes=[pltpu.VMEM((num_experts, blk), jnp.float32)],
    )(xT)


def _sc_hist(idxT, num_experts):
    width = idxT.shape[1]
    vecs = width // _L
    mesh = plsc.VectorSubcoreMesh(core_axis_name="c", subcore_axis_name="s")
    cp = pltpu.CompilerParams()
    if "needs_layout_passes" in pltpu.CompilerParams.__dataclass_fields__:
        cp = dataclasses.replace(cp, needs_layout_passes=False)

    @functools.partial(
        pl.kernel,
        mesh=mesh,
        compiler_params=cp,
        out_type=jax.ShapeDtypeStruct((_NW * _L * num_experts,), jnp.float32),
        scratch_types=[
            pltpu.VMEM((width,), jnp.int32),
            pltpu.VMEM((_L * num_experts,), jnp.float32),
            pltpu.SemaphoreType.DMA,
        ],
    )
    def hist_kernel(idx_hbm, out_hbm, idx_v, hist_v, sem):
        c = lax.axis_index("c")
        s = lax.axis_index("s")
        wid = s * _NC + c
        cpy = pltpu.async_copy(idx_hbm.at[wid, pl.ds(0, width)], idx_v, sem)

        zeros = jnp.zeros((_L,), jnp.float32)
        for j in range(num_experts):
            hist_v[pl.ds(j * _L, _L)] = zeros

        lane_off = lax.broadcasted_iota(jnp.int32, (_L,), 0) * num_experts
        ones = jnp.ones((_L,), jnp.float32)
        cpy.wait()

        @pl.loop(0, vecs, step=8)
        def _(i):
            for j in range(8):
                e = idx_v[pl.ds((i + j) * _L, _L)]
                plsc.addupdate_scatter(hist_v, [lane_off + e], ones)

        pltpu.sync_copy(hist_v, out_hbm.at[pl.ds(wid * _L * num_experts,
                                                 _L * num_experts)])

    return hist_kernel(idxT)


def _combine_body(psum_ref, cnt_ref, out_ref, *, inv_tokens, inv_assign,
                  num_experts):
    ne = int(num_experts)
    cnt = jnp.sum(cnt_ref[...], axis=0, keepdims=True)
    cnt64 = cnt[:, :ne] + cnt[:, ne:]
    avg_prob = psum_ref[...].T * inv_tokens
    frac = cnt64 * inv_assign
    out_ref[0] = 0.01 * num_experts * jnp.sum(frac * avg_prob)


def _combine(psum, cnt2d, tokens, total_assign):
    num_experts = psum.shape[0]
    body = functools.partial(
        _combine_body,
        inv_tokens=1.0 / tokens,
        inv_assign=1.0 / total_assign,
        num_experts=float(num_experts),
    )
    return pl.pallas_call(
        body,
        out_specs=pl.BlockSpec(memory_space=pltpu.SMEM),
        out_shape=jax.ShapeDtypeStruct((1,), jnp.float32),
    )(psum, cnt2d)


def kernel(router_logits, expert_indices):
    batch, seq, num_experts = router_logits.shape
    k = expert_indices.shape[-1]
    tokens = batch * seq
    total_assign = tokens * k

    xT = jnp.transpose(router_logits, (0, 2, 1)).reshape(batch * num_experts,
                                                         seq)
    idxT = jnp.transpose(expert_indices, (0, 2, 1)).reshape(
        batch * k, seq).astype(jnp.int32)

    psum = _softmax_psum(xT, num_experts)
    counts_flat = _sc_hist(idxT, num_experts)
    cnt2d = counts_flat.reshape(_NW * _L * num_experts // 128, 128)

    loss = _combine(psum, cnt2d, tokens, total_assign)
    return loss[0]


# speedup vs baseline: 2.7160x; 1.0017x over previous
import dataclasses
import functools

import jax
import jax.numpy as jnp
from jax import lax
from jax.experimental import pallas as pl
from jax.experimental.pallas import tpu as pltpu
from jax.experimental.pallas import tpu_sc as plsc

_NC = 2
_NS = 16
_NW = _NC * _NS
_L = 16


def _softmax_body(x_ref, psum_ref, acc_ref):
    i = pl.program_id(0)
    j = pl.program_id(1)

    @pl.when((i == 0) & (j == 0))
    def _():
        acc_ref[...] = jnp.zeros_like(acc_ref)

    x = x_ref[...]
    m = jnp.max(x, axis=0, keepdims=True)
    e = jnp.exp(x - m)
    r = 1.0 / jnp.sum(e, axis=0, keepdims=True)
    acc_ref[...] += e * r

    @pl.when((i == pl.num_programs(0) - 1) & (j == pl.num_programs(1) - 1))
    def _():
        psum_ref[...] = jnp.sum(acc_ref[...], axis=1, keepdims=True)


def _softmax_psum(xT, num_experts):
    rows, seq = xT.shape
    batch = rows // num_experts
    blk = 4096
    return pl.pallas_call(
        _softmax_body,
        grid=(batch, seq // blk),
        in_specs=[pl.BlockSpec((num_experts, blk), lambda i, j: (i, j))],
        out_specs=pl.BlockSpec((num_experts, 1), lambda i, j: (0, 0)),
        out_shape=jax.ShapeDtypeStruct((num_experts, 1), jnp.float32),
        scratch_shapes=[pltpu.VMEM((num_experts, blk), jnp.float32)],
    )(xT)


def _sc_hist(idxT, num_experts):
    width = idxT.shape[1]
    vecs = width // _L
    mesh = plsc.VectorSubcoreMesh(core_axis_name="c", subcore_axis_name="s")
    cp = pltpu.CompilerParams()
    if "needs_layout_passes" in pltpu.CompilerParams.__dataclass_fields__:
        cp = dataclasses.replace(cp, needs_layout_passes=False)

    @functools.partial(
        pl.kernel,
        mesh=mesh,
        compiler_params=cp,
        out_type=jax.ShapeDtypeStruct((_NW * _L * num_experts,), jnp.float32),
        scratch_types=[
            pltpu.VMEM((width,), jnp.int32),
            pltpu.VMEM((_L * num_experts,), jnp.float32),
            pltpu.SemaphoreType.DMA,
        ],
    )
    def hist_kernel(idx_hbm, out_hbm, idx_v, hist_v, sem):
        c = lax.axis_index("c")
        s = lax.axis_index("s")
        wid = s * _NC + c
        cpy = pltpu.async_copy(idx_hbm.at[wid, pl.ds(0, width)], idx_v, sem)

        zeros = jnp.zeros((_L,), jnp.float32)

        @pl.loop(0, num_experts)
        def _(j):
            hist_v[pl.ds(j * _L, _L)] = zeros

        lane_off = lax.broadcasted_iota(jnp.int32, (_L,), 0) * num_experts
        ones = jnp.ones((_L,), jnp.float32)
        cpy.wait()

        @pl.loop(0, vecs)
        def _(i):
            e = idx_v[pl.ds(i * _L, _L)]
            plsc.addupdate_scatter(hist_v, [lane_off + e], ones)

        pltpu.sync_copy(hist_v, out_hbm.at[pl.ds(wid * _L * num_experts,
                                                 _L * num_experts)])

    return hist_kernel(idxT)


def _combine_body(psum_ref, cnt_ref, out_ref, *, inv_tokens, inv_assign,
                  num_experts):
    ne = int(num_experts)
    cnt = jnp.sum(cnt_ref[...], axis=0, keepdims=True)
    cnt64 = cnt[:, :ne] + cnt[:, ne:]
    avg_prob = psum_ref[...].T * inv_tokens
    frac = cnt64 * inv_assign
    out_ref[0] = 0.01 * num_experts * jnp.sum(frac * avg_prob)


def _combine(psum, cnt2d, tokens, total_assign):
    num_experts = psum.shape[0]
    body = functools.partial(
        _combine_body,
        inv_tokens=1.0 / tokens,
        inv_assign=1.0 / total_assign,
        num_experts=float(num_experts),
    )
    return pl.pallas_call(
        body,
        out_specs=pl.BlockSpec(memory_space=pltpu.SMEM),
        out_shape=jax.ShapeDtypeStruct((1,), jnp.float32),
    )(psum, cnt2d)


def kernel(router_logits, expert_indices):
    batch, seq, num_experts = router_logits.shape
    k = expert_indices.shape[-1]
    tokens = batch * seq
    total_assign = tokens * k

    xT = jnp.transpose(router_logits, (0, 2, 1)).reshape(batch * num_experts,
                                                         seq)
    idxT = jnp.transpose(expert_indices, (0, 2, 1)).reshape(
        batch * k, seq).astype(jnp.int32)

    psum = _softmax_psum(xT, num_experts)
    counts_flat = _sc_hist(idxT, num_experts)
    cnt2d = counts_flat.reshape(_NW * _L * num_experts // 128, 128)

    loss = _combine(psum, cnt2d, tokens, total_assign)
    return loss[0]
